# baseline (speedup 1.0000x reference)
import functools

import jax
import jax.numpy as jnp
from jax import lax
from jax.experimental import pallas as pl
from jax.experimental.pallas import tpu as pltpu
from jax.experimental.pallas import tpu_sc as plsc

B = 16384
L = 16
TOP = 1111111
M = 1000000

_info = plsc.get_sparse_core_info()
NC = _info.num_cores
NS = _info.num_subcores
NW = NC * NS
BPW = B // NW


HALF = 256


def _compute_range(n_v, o_v, out_v, lo_idx, hi_idx):
    for i in range(lo_idx, hi_idx):
        nv = n_v[pl.ds(i * L, L)]
        ov = o_v[pl.ds(i * L, L)]
        r = nv - TOP
        for t in (8 * M, 4 * M, 2 * M, M):
            r = jnp.where(r >= t, r - t, r)
        cond = nv >= TOP
        m = jnp.where(cond, r, nv)
        c = jnp.where(cond, TOP, 1)
        out_v[pl.ds(i * L, L)] = m * 10 + ov + c


def _sc_body(n_hbm, o_hbm, out_hbm, n_v, o_v, out_v,
             sem_n1, sem_o1, sem_n2, sem_o2, sem_out):
    wid = lax.axis_index("s") * NC + lax.axis_index("c")
    start = wid * BPW
    cp_n1 = pltpu.async_copy(
        n_hbm.at[pl.ds(start, HALF)], n_v.at[pl.ds(0, HALF)], sem_n1)
    cp_o1 = pltpu.async_copy(
        o_hbm.at[pl.ds(start, HALF)], o_v.at[pl.ds(0, HALF)], sem_o1)
    cp_n2 = pltpu.async_copy(
        n_hbm.at[pl.ds(start + HALF, HALF)], n_v.at[pl.ds(HALF, HALF)], sem_n2)
    cp_o2 = pltpu.async_copy(
        o_hbm.at[pl.ds(start + HALF, HALF)], o_v.at[pl.ds(HALF, HALF)], sem_o2)
    cp_n1.wait()
    cp_o1.wait()
    _compute_range(n_v, o_v, out_v, 0, HALF // L)
    cp_out1 = pltpu.async_copy(
        out_v.at[pl.ds(0, HALF)], out_hbm.at[pl.ds(start, HALF)], sem_out)
    cp_n2.wait()
    cp_o2.wait()
    _compute_range(n_v, o_v, out_v, HALF // L, BPW // L)
    cp_out1.wait()
    pltpu.sync_copy(
        out_v.at[pl.ds(HALF, HALF)], out_hbm.at[pl.ds(start + HALF, HALF)])


_sc_call = functools.partial(
    pl.kernel,
    mesh=plsc.VectorSubcoreMesh(
        core_axis_name="c", subcore_axis_name="s", num_cores=NC),
    out_type=jax.ShapeDtypeStruct((B,), jnp.int32),
    scratch_types=[
        pltpu.VMEM((BPW,), jnp.int32),
        pltpu.VMEM((BPW,), jnp.int32),
        pltpu.VMEM((BPW,), jnp.int32),
        pltpu.SemaphoreType.DMA,
        pltpu.SemaphoreType.DMA,
        pltpu.SemaphoreType.DMA,
        pltpu.SemaphoreType.DMA,
        pltpu.SemaphoreType.DMA,
    ],
)(_sc_body)


def kernel(n, o):
    n1 = _sc_call(n, o)
    return (n1, jnp.zeros((), dtype=n1.dtype))


# speedup vs baseline: 1.0330x; 1.0330x over previous
import functools

import jax
import jax.numpy as jnp
from jax import lax
from jax.experimental import pallas as pl
from jax.experimental.pallas import tpu as pltpu
from jax.experimental.pallas import tpu_sc as plsc

B = 16384
L = 16
TOP = 1111111
M = 1000000

_info = plsc.get_sparse_core_info()
NC = 1
NS = _info.num_subcores
NW = NC * NS
BPW = B // NW


HALF = 512


def _compute_range(n_v, o_v, out_v, lo_idx, hi_idx):
    def step(i, carry):
        off = pl.multiple_of(i * L, L)
        nv = n_v[pl.ds(off, L)]
        ov = o_v[pl.ds(off, L)]
        r = nv - TOP
        for t in (8 * M, 4 * M, 2 * M, M):
            r = jnp.where(r >= t, r - t, r)
        cond = nv >= TOP
        m = jnp.where(cond, r, nv)
        c = jnp.where(cond, TOP, 1)
        out_v[pl.ds(off, L)] = m * 10 + ov + c
        return carry

    lax.fori_loop(lo_idx, hi_idx, step, 0, unroll=4)


def _sc_body(n_hbm, o_hbm, out_hbm, n_v, o_v, out_v,
             sem_n1, sem_o1, sem_n2, sem_o2, sem_out):
    wid = lax.axis_index("s") * NC + lax.axis_index("c")
    start = wid * BPW
    cp_n1 = pltpu.async_copy(
        n_hbm.at[pl.ds(start, HALF)], n_v.at[pl.ds(0, HALF)], sem_n1)
    cp_o1 = pltpu.async_copy(
        o_hbm.at[pl.ds(start, HALF)], o_v.at[pl.ds(0, HALF)], sem_o1)
    cp_n2 = pltpu.async_copy(
        n_hbm.at[pl.ds(start + HALF, HALF)], n_v.at[pl.ds(HALF, HALF)], sem_n2)
    cp_o2 = pltpu.async_copy(
        o_hbm.at[pl.ds(start + HALF, HALF)], o_v.at[pl.ds(HALF, HALF)], sem_o2)
    cp_n1.wait()
    cp_o1.wait()
    _compute_range(n_v, o_v, out_v, 0, HALF // L)
    cp_out1 = pltpu.async_copy(
        out_v.at[pl.ds(0, HALF)], out_hbm.at[pl.ds(start, HALF)], sem_out)
    cp_n2.wait()
    cp_o2.wait()
    _compute_range(n_v, o_v, out_v, HALF // L, BPW // L)
    cp_out1.wait()
    pltpu.sync_copy(
        out_v.at[pl.ds(HALF, HALF)], out_hbm.at[pl.ds(start + HALF, HALF)])


_sc_call = functools.partial(
    pl.kernel,
    mesh=plsc.VectorSubcoreMesh(
        core_axis_name="c", subcore_axis_name="s", num_cores=NC),
    out_type=jax.ShapeDtypeStruct((B,), jnp.int32),
    scratch_types=[
        pltpu.VMEM((BPW,), jnp.int32),
        pltpu.VMEM((BPW,), jnp.int32),
        pltpu.VMEM((BPW,), jnp.int32),
        pltpu.SemaphoreType.DMA,
        pltpu.SemaphoreType.DMA,
        pltpu.SemaphoreType.DMA,
        pltpu.SemaphoreType.DMA,
        pltpu.SemaphoreType.DMA,
    ],
)(_sc_body)


def kernel(n, o):
    n1 = _sc_call(n, o)
    return (n1, jnp.zeros((), dtype=n1.dtype))
